# baseline (speedup 1.0000x reference)
import functools

import jax
import jax.numpy as jnp
from jax import lax
from jax.experimental import pallas as pl
from jax.experimental.pallas import tpu as pltpu
from jax.experimental.pallas import tpu_sc as plsc

VOCAB_ROWS = 32
EMBED_DIM = 64
BATCH = 4096
SEQ = 200
TOTAL = BATCH * SEQ

_info = plsc.get_sparse_core_info()
_NC = _info.num_cores
_NS = _info.num_subcores
_NW = _NC * _NS
PER_W = TOTAL // _NW
CHUNK = 128
N_CHUNKS = PER_W // CHUNK
GPB = 2
BUF_ROWS = GPB * CHUNK
NSTEP = N_CHUNKS // GPB
NBUF = 4
SKEW = 2


def _make_kernel():
    mesh = plsc.VectorSubcoreMesh(core_axis_name="c", subcore_axis_name="s")

    @functools.partial(
        pl.kernel,
        mesh=mesh,
        out_type=jax.ShapeDtypeStruct((TOTAL, EMBED_DIM), jnp.float32),
        compiler_params=pltpu.CompilerParams(use_tc_tiling_on_sc=False),
        scratch_types=[
            pltpu.VMEM((N_CHUNKS, CHUNK), jnp.int32),
            pltpu.VMEM((NBUF, BUF_ROWS, EMBED_DIM), jnp.float32),
            pltpu.VMEM_SHARED((VOCAB_ROWS, EMBED_DIM), jnp.float32),
        ]
        + [pltpu.SemaphoreType.DMA] * (2 * NBUF),
    )
    def k(idx_hbm, table_hbm, out_hbm, idx_v, rows, table_sh,
          g0, g1, g2, g3, o0, o1, o2, o3):
        gsem = [g0, g1, g2, g3]
        osem = [o0, o1, o2, o3]
        sid = lax.axis_index("s")
        wid = sid * _NC + lax.axis_index("c")
        base = wid * PER_W

        @pl.when(sid == 0)
        def _():
            pltpu.sync_copy(table_hbm, table_sh)

        plsc.subcore_barrier()

        pltpu.sync_copy(idx_hbm.at[pl.ds(wid * N_CHUNKS, N_CHUNKS)], idx_v)

        def gathers(s, b, start):
            for j in range(GPB):
                cp = pltpu.make_async_copy(
                    table_sh.at[idx_v.at[s * GPB + j]],
                    rows.at[b, pl.ds(j * CHUNK, CHUNK)],
                    gsem[b],
                )
                cp.start() if start else cp.wait()

        def write(s, b, start):
            cp = pltpu.make_async_copy(
                rows.at[b],
                out_hbm.at[pl.ds(base + s * BUF_ROWS, BUF_ROWS)],
                osem[b],
            )
            cp.start() if start else cp.wait()

        for b in range(NBUF):
            gathers(b, b, True)
        for b in range(SKEW):
            gathers(b, b, False)
            write(b, b, True)

        def body(i, carry):
            sbase = i * NBUF
            for b in range(NBUF):
                s = sbase + b
                write(s - NBUF, b, False)
                gathers(s, b, True)
                sw = s - SKEW
                bw = (b + NBUF - SKEW) % NBUF
                gathers(sw, bw, False)
                write(sw, bw, True)
            return carry

        lax.fori_loop(1, NSTEP // NBUF, body, 0)

        last = NSTEP - NBUF
        for b in range(SKEW, NBUF):
            s = last + b
            gathers(s, b, False)
            write(s, b, True)
        for b in range(NBUF):
            write(last + b, b, False)

    return k


_sc_gather = _make_kernel()


def kernel(inputs, embedding_table):
    idx = inputs.reshape(TOTAL // CHUNK, CHUNK)
    out = _sc_gather(idx, embedding_table)
    return out.reshape(BATCH, SEQ, EMBED_DIM)
